# baseline (speedup 1.0000x reference)
import functools

import jax
import jax.numpy as jnp
from jax import lax
from jax.experimental import pallas as pl
from jax.experimental.pallas import tpu as pltpu
from jax.experimental.pallas import tpu_sc as plsc

BATCH = 16384
HIST = 50
EMBED_DIM = 32
TOTAL = BATCH * HIST
VOCAB = 1000000

NUM_CORES = 2
NUM_WORKERS = 32

CHUNK = 512
CHUNKS_PER_H = BATCH // CHUNK
NUM_UNITS = TOTAL // CHUNK
UNITS_PER_WORKER = NUM_UNITS // NUM_WORKERS

_mesh = plsc.VectorSubcoreMesh(core_axis_name="c", subcore_axis_name="s")


@functools.partial(
    pl.kernel,
    mesh=_mesh,
    out_type=jax.ShapeDtypeStruct((HIST, EMBED_DIM, BATCH), jnp.float32),
    scratch_types=[
        pltpu.VMEM((CHUNK,), jnp.int32),
        pltpu.VMEM((CHUNK,), jnp.int32),
        pltpu.VMEM((CHUNK,), jnp.int32),
        pltpu.VMEM((CHUNK,), jnp.int32),
        pltpu.VMEM((CHUNK, EMBED_DIM), jnp.float32),
        pltpu.VMEM((CHUNK, EMBED_DIM), jnp.float32),
        pltpu.VMEM((EMBED_DIM * CHUNK,), jnp.float32),
        pltpu.VMEM((EMBED_DIM * CHUNK,), jnp.float32),
        pltpu.SemaphoreType.DMA,
        pltpu.SemaphoreType.DMA,
        pltpu.SemaphoreType.DMA,
        pltpu.SemaphoreType.DMA,
        pltpu.SemaphoreType.DMA,
    ],
    compiler_params=pltpu.CompilerParams(
        use_tc_tiling_on_sc=False, needs_layout_passes=False),
)
def _gather_kernel(idx_hbm, table_hbm, out_hbm, posa, posb, idxa, idxb,
                   rowsa, rowsb, ta, tb, psema, psemb, gsema, gsemb, osem):
    wid = lax.axis_index("s") * NUM_CORES + lax.axis_index("c")
    iota16 = jax.lax.iota(jnp.int32, 16)
    iota_sc = iota16 * CHUNK
    iota_h = iota16 * HIST

    def fetch(k, pos_v, idx_v, rows_v, psem, gsem):
        u = k * NUM_WORKERS + wid
        h = u // CHUNKS_PER_H
        b0 = (u % CHUNKS_PER_H) * CHUNK
        p0 = b0 * HIST + h
        for g in range(CHUNK // 16):
            pos_v[pl.ds(g * 16, 16)] = iota_h + (p0 + g * 16 * HIST)
        pltpu.async_copy(idx_hbm.at[pos_v], idx_v, psem).wait()
        return pltpu.async_copy(table_hbm.at[idx_v], rows_v, gsem)

    def process(k, rows_v, tbuf, gsem):
        pltpu.make_async_copy(table_hbm.at[pl.ds(0, CHUNK)], rows_v,
                              gsem).wait()
        for r in range(CHUNK):
            for q in range(EMBED_DIM // 16):
                vals = rows_v[r, pl.ds(q * 16, 16)]
                plsc.store_scatter(
                    tbuf, [iota_sc + (q * 16 * CHUNK + r)], vals)
        u = k * NUM_WORKERS + wid
        h = u // CHUNKS_PER_H
        b0 = (u % CHUNKS_PER_H) * CHUNK
        copies = [
            pltpu.async_copy(
                tbuf.at[pl.ds(c * CHUNK, CHUNK)],
                out_hbm.at[h, c, pl.ds(b0, CHUNK)],
                osem,
            )
            for c in range(EMBED_DIM)
        ]
        for cp in copies:
            cp.wait()

    fetch(0, posa, idxa, rowsa, psema, gsema)

    def body(k2, carry):
        ka = 2 * k2
        fetch(ka + 1, posb, idxb, rowsb, psemb, gsemb)
        process(ka, rowsa, ta, gsema)

        @pl.when(k2 + 1 < UNITS_PER_WORKER // 2)
        def _():
            fetch(ka + 2, posa, idxa, rowsa, psema, gsema)

        process(ka + 1, rowsb, tb, gsemb)
        return carry

    lax.fori_loop(0, UNITS_PER_WORKER // 2, body, 0)


def kernel(inputs, table):
    idx_flat = inputs.reshape(TOTAL)
    out = _gather_kernel(idx_flat, table)
    return out.transpose(2, 0, 1)


# speedup vs baseline: 1.3508x; 1.3508x over previous
import functools

import jax
import jax.numpy as jnp
from jax import lax
from jax.experimental import pallas as pl
from jax.experimental.pallas import tpu as pltpu
from jax.experimental.pallas import tpu_sc as plsc

BATCH = 16384
HIST = 50
EMBED_DIM = 32
TOTAL = BATCH * HIST
VOCAB = 1000000

NUM_CORES = 2
NUM_WORKERS = 32

CHUNK = 512
CHUNKS_PER_H = BATCH // CHUNK
NUM_UNITS = TOTAL // CHUNK
UNITS_PER_WORKER = NUM_UNITS // NUM_WORKERS
TPAD = CHUNK + 8

_mesh = plsc.VectorSubcoreMesh(core_axis_name="c", subcore_axis_name="s")


@functools.partial(
    pl.kernel,
    mesh=_mesh,
    out_type=jax.ShapeDtypeStruct((TOTAL,), jnp.int32),
    scratch_types=[
        pltpu.VMEM((BATCH,), jnp.int32),
    ],
)
def _idx_kernel(idxt_hbm, flat_hbm, vbuf):
    wid = lax.axis_index("s") * NUM_CORES + lax.axis_index("c")

    def move(h):
        pltpu.sync_copy(idxt_hbm.at[h, :], vbuf)
        pltpu.sync_copy(vbuf, flat_hbm.at[pl.ds(h * BATCH, BATCH)])

    move(wid)

    @pl.when(wid < HIST - NUM_WORKERS)
    def _():
        move(wid + NUM_WORKERS)


@functools.partial(
    pl.kernel,
    mesh=_mesh,
    out_type=jax.ShapeDtypeStruct((HIST, EMBED_DIM, BATCH), jnp.float32),
    scratch_types=[
        pltpu.VMEM((CHUNK,), jnp.int32),
        pltpu.VMEM((CHUNK,), jnp.int32),
        pltpu.VMEM((CHUNK,), jnp.int32),
        pltpu.VMEM((CHUNK,), jnp.int32),
        pltpu.VMEM((CHUNK, EMBED_DIM), jnp.float32),
        pltpu.VMEM((CHUNK, EMBED_DIM), jnp.float32),
        pltpu.VMEM((EMBED_DIM * TPAD,), jnp.float32),
        pltpu.VMEM((EMBED_DIM * TPAD,), jnp.float32),
        pltpu.SemaphoreType.DMA,
        pltpu.SemaphoreType.DMA,
        pltpu.SemaphoreType.DMA,
        pltpu.SemaphoreType.DMA,
        pltpu.SemaphoreType.DMA,
    ],
    compiler_params=pltpu.CompilerParams(
        use_tc_tiling_on_sc=False, needs_layout_passes=False),
)
def _gather_kernel(idx_hbm, table_hbm, out_hbm, posa, posb, idxa, idxb,
                   rowsa, rowsb, ta, tb, psema, psemb, gsema, gsemb, osem):
    wid = lax.axis_index("s") * NUM_CORES + lax.axis_index("c")
    iota16 = jax.lax.iota(jnp.int32, 16)
    iota_sc = iota16 * TPAD
    iota_h = iota16 * HIST

    def fetch(k, pos_v, idx_v, rows_v, psem, gsem):
        off = (k * NUM_WORKERS + wid) * CHUNK
        pltpu.sync_copy(idx_hbm.at[pl.ds(off, CHUNK)], idx_v)
        return pltpu.async_copy(table_hbm.at[idx_v], rows_v, gsem)

    def process(k, rows_v, tbuf, gsem):
        pltpu.make_async_copy(table_hbm.at[pl.ds(0, CHUNK)], rows_v,
                              gsem).wait()
        for r in range(CHUNK):
            for q in range(EMBED_DIM // 16):
                vals = rows_v[r, pl.ds(q * 16, 16)]
                plsc.store_scatter(
                    tbuf, [iota_sc + (q * 16 * TPAD + r)], vals)
        u = k * NUM_WORKERS + wid
        h = u // CHUNKS_PER_H
        b0 = (u % CHUNKS_PER_H) * CHUNK
        copies = [
            pltpu.async_copy(
                tbuf.at[pl.ds(c * TPAD, CHUNK)],
                out_hbm.at[h, c, pl.ds(b0, CHUNK)],
                osem,
            )
            for c in range(EMBED_DIM)
        ]
        for cp in copies:
            cp.wait()

    fetch(0, posa, idxa, rowsa, psema, gsema)

    def body(k2, carry):
        ka = 2 * k2
        fetch(ka + 1, posb, idxb, rowsb, psemb, gsemb)
        process(ka, rowsa, ta, gsema)

        @pl.when(k2 + 1 < UNITS_PER_WORKER // 2)
        def _():
            fetch(ka + 2, posa, idxa, rowsa, psema, gsema)

        process(ka + 1, rowsb, tb, gsemb)
        return carry

    lax.fori_loop(0, UNITS_PER_WORKER // 2, body, 0)


def kernel(inputs, table):
    idx_h_major = _idx_kernel(inputs.T)
    out = _gather_kernel(idx_h_major, table)
    return out.transpose(2, 0, 1)


# speedup vs baseline: 1.6887x; 1.2502x over previous
import functools

import jax
import jax.numpy as jnp
from jax import lax
from jax.experimental import pallas as pl
from jax.experimental.pallas import tpu as pltpu
from jax.experimental.pallas import tpu_sc as plsc

BATCH = 16384
HIST = 50
EMBED_DIM = 32
TOTAL = BATCH * HIST
VOCAB = 1000000

NUM_CORES = 2
NUM_WORKERS = 32

CHUNK = 512
CHUNKS_PER_H = BATCH // CHUNK
NUM_UNITS = TOTAL // CHUNK
UNITS_PER_WORKER = NUM_UNITS // NUM_WORKERS
TPAD = CHUNK + 8

_mesh = plsc.VectorSubcoreMesh(core_axis_name="c", subcore_axis_name="s")


@functools.partial(
    pl.kernel,
    mesh=_mesh,
    out_type=jax.ShapeDtypeStruct((TOTAL,), jnp.int32),
    scratch_types=[
        pltpu.VMEM((BATCH,), jnp.int32),
    ],
)
def _idx_kernel(idxt_hbm, flat_hbm, vbuf):
    wid = lax.axis_index("s") * NUM_CORES + lax.axis_index("c")

    def move(h):
        pltpu.sync_copy(idxt_hbm.at[h, :], vbuf)
        pltpu.sync_copy(vbuf, flat_hbm.at[pl.ds(h * BATCH, BATCH)])

    move(wid)

    @pl.when(wid < HIST - NUM_WORKERS)
    def _():
        move(wid + NUM_WORKERS)


@functools.partial(
    pl.kernel,
    mesh=_mesh,
    out_type=jax.ShapeDtypeStruct((HIST, EMBED_DIM, BATCH), jnp.float32),
    scratch_types=[
        pltpu.VMEM((CHUNK,), jnp.int32),
        pltpu.VMEM((CHUNK,), jnp.int32),
        pltpu.VMEM((CHUNK,), jnp.int32),
        pltpu.VMEM((CHUNK,), jnp.int32),
        pltpu.VMEM((CHUNK, EMBED_DIM), jnp.float32),
        pltpu.VMEM((CHUNK, EMBED_DIM), jnp.float32),
        pltpu.VMEM((EMBED_DIM * TPAD,), jnp.float32),
        pltpu.VMEM((EMBED_DIM * TPAD,), jnp.float32),
        pltpu.SemaphoreType.DMA,
        pltpu.SemaphoreType.DMA,
        pltpu.SemaphoreType.DMA,
        pltpu.SemaphoreType.DMA,
        pltpu.SemaphoreType.DMA,
    ],
    compiler_params=pltpu.CompilerParams(
        use_tc_tiling_on_sc=False, needs_layout_passes=False),
)
def _gather_kernel(idx_hbm, table_hbm, out_hbm, posa, posb, idxa, idxb,
                   rowsa, rowsb, ta, tb, psema, psemb, gsema, gsemb, osem):
    wid = lax.axis_index("s") * NUM_CORES + lax.axis_index("c")
    iota16 = jax.lax.iota(jnp.int32, 16)
    iota_sc = iota16 * TPAD
    iota_h = iota16 * HIST

    def fetch(k, pos_v, idx_v, rows_v, psem, gsem):
        off = (k * NUM_WORKERS + wid) * CHUNK
        pltpu.sync_copy(idx_hbm.at[pl.ds(off, CHUNK)], idx_v)
        return pltpu.async_copy(table_hbm.at[idx_v], rows_v, gsem)

    def process(k, rows_v, tbuf, gsem):
        pltpu.make_async_copy(table_hbm.at[pl.ds(0, CHUNK)], rows_v,
                              gsem).wait()
        @plsc.parallel_loop(0, CHUNK, 1, unroll=8)
        def _transpose(r):
            for q in range(EMBED_DIM // 16):
                vals = rows_v[r, pl.ds(q * 16, 16)]
                plsc.store_scatter(
                    tbuf, [iota_sc + (q * 16 * TPAD) + r], vals)
        u = k * NUM_WORKERS + wid
        h = u // CHUNKS_PER_H
        b0 = (u % CHUNKS_PER_H) * CHUNK
        copies = [
            pltpu.async_copy(
                tbuf.at[pl.ds(c * TPAD, CHUNK)],
                out_hbm.at[h, c, pl.ds(b0, CHUNK)],
                osem,
            )
            for c in range(EMBED_DIM)
        ]
        for cp in copies:
            cp.wait()

    fetch(0, posa, idxa, rowsa, psema, gsema)

    def body(k2, carry):
        ka = 2 * k2
        fetch(ka + 1, posb, idxb, rowsb, psemb, gsemb)
        process(ka, rowsa, ta, gsema)

        @pl.when(k2 + 1 < UNITS_PER_WORKER // 2)
        def _():
            fetch(ka + 2, posa, idxa, rowsa, psema, gsema)

        process(ka + 1, rowsb, tb, gsemb)
        return carry

    lax.fori_loop(0, UNITS_PER_WORKER // 2, body, 0)


def kernel(inputs, table):
    idx_h_major = _idx_kernel(inputs.T)
    out = _gather_kernel(idx_h_major, table)
    return out.transpose(2, 0, 1)


# speedup vs baseline: 2.4931x; 1.4764x over previous
import functools

import jax
import jax.numpy as jnp
from jax import lax
from jax.experimental import pallas as pl
from jax.experimental.pallas import tpu as pltpu
from jax.experimental.pallas import tpu_sc as plsc

BATCH = 16384
HIST = 50
EMBED_DIM = 32
TOTAL = BATCH * HIST
VOCAB = 1000000

NUM_CORES = 2
NUM_WORKERS = 32

CHUNK = 512
CHUNKS_PER_H = BATCH // CHUNK
NUM_UNITS = TOTAL // CHUNK
UNITS_PER_WORKER = NUM_UNITS // NUM_WORKERS
TPAD = CHUNK + 8

_mesh = plsc.VectorSubcoreMesh(core_axis_name="c", subcore_axis_name="s")


TBLK = 512
NBLK = VOCAB // TBLK
GRP = 1024 + 8


@functools.partial(
    pl.kernel,
    mesh=_mesh,
    out_type=jax.ShapeDtypeStruct((VOCAB * EMBED_DIM,), jnp.float32),
    scratch_types=[
        pltpu.VMEM((EMBED_DIM, TBLK), jnp.float32),
        pltpu.VMEM((16 * GRP,), jnp.float32),
        pltpu.SemaphoreType.DMA,
    ],
    compiler_params=pltpu.CompilerParams(needs_layout_passes=False),
)
def _table_kernel(tt_hbm, flat_hbm, cbuf2d, rbuf, dsem):
    wid = lax.axis_index("s") * NUM_CORES + lax.axis_index("c")
    iota_pb = jax.lax.iota(jnp.int32, 16) * GRP

    def transpose_block(ngrp):
        @plsc.parallel_loop(0, ngrp, 1, unroll=2)
        def _t(g):
            for c in range(EMBED_DIM):
                vals = cbuf2d[c, pl.ds(g * 16, 16)]
                plsc.store_scatter(rbuf, [iota_pb + (g * EMBED_DIM + c)],
                                   vals)

    def writeback(r0, nwords):
        for k in range(16):
            pltpu.sync_copy(
                rbuf.at[pl.ds(k * GRP, nwords)],
                flat_hbm.at[pl.ds(r0 * EMBED_DIM + k * 1024, nwords)])

    def body(k, carry):
        j = k * NUM_WORKERS + wid

        @pl.when(j < NBLK)
        def _():
            r0 = j * TBLK
            for kb in range(EMBED_DIM // 8):
                pltpu.async_copy(tt_hbm.at[pl.ds(8 * kb, 8), pl.ds(r0, TBLK)],
                                 cbuf2d.at[pl.ds(8 * kb, 8), :], dsem)
            pltpu.make_async_copy(
                flat_hbm.at[pl.ds(0, EMBED_DIM * TBLK)],
                rbuf.at[pl.ds(0, EMBED_DIM * TBLK)], dsem).wait()
            transpose_block(TBLK // 16)
            writeback(r0, 1024)

        return carry

    lax.fori_loop(0, (NBLK + NUM_WORKERS - 1) // NUM_WORKERS, body, 0)

    @pl.when(wid == NBLK % NUM_WORKERS)
    def _tail():
        r0 = NBLK * TBLK
        tail = VOCAB - r0
        for c in range(EMBED_DIM):
            pltpu.async_copy(tt_hbm.at[c, pl.ds(r0, tail)],
                             cbuf2d.at[c, pl.ds(0, tail)], dsem)
        pltpu.make_async_copy(
            flat_hbm.at[pl.ds(0, EMBED_DIM * tail)],
            rbuf.at[pl.ds(0, EMBED_DIM * tail)], dsem).wait()

        iota32 = jax.lax.iota(jnp.int32, 16) * EMBED_DIM

        @plsc.parallel_loop(0, tail // 16, 1)
        def _tt(g):
            for c in range(EMBED_DIM):
                vals = cbuf2d[c, pl.ds(g * 16, 16)]
                plsc.store_scatter(
                    rbuf, [iota32 + (g * 16 * EMBED_DIM + c)], vals)

        pltpu.sync_copy(rbuf.at[pl.ds(0, tail * EMBED_DIM)],
                        flat_hbm.at[pl.ds(r0 * EMBED_DIM,
                                          tail * EMBED_DIM)])


@functools.partial(
    pl.kernel,
    mesh=_mesh,
    out_type=jax.ShapeDtypeStruct((TOTAL,), jnp.int32),
    scratch_types=[
        pltpu.VMEM((BATCH,), jnp.int32),
    ],
)
def _idx_kernel(idxt_hbm, flat_hbm, vbuf):
    wid = lax.axis_index("s") * NUM_CORES + lax.axis_index("c")

    def move(h):
        pltpu.sync_copy(idxt_hbm.at[h, :], vbuf)
        pltpu.sync_copy(vbuf, flat_hbm.at[pl.ds(h * BATCH, BATCH)])

    move(wid)

    @pl.when(wid < HIST - NUM_WORKERS)
    def _():
        move(wid + NUM_WORKERS)


@functools.partial(
    pl.kernel,
    mesh=_mesh,
    out_type=jax.ShapeDtypeStruct((HIST, EMBED_DIM, BATCH), jnp.float32),
    scratch_types=[
        pltpu.VMEM((CHUNK,), jnp.int32),
        pltpu.VMEM((CHUNK,), jnp.int32),
        pltpu.VMEM((CHUNK,), jnp.int32),
        pltpu.VMEM((CHUNK,), jnp.int32),
        pltpu.VMEM((CHUNK, EMBED_DIM), jnp.float32),
        pltpu.VMEM((CHUNK, EMBED_DIM), jnp.float32),
        pltpu.VMEM((EMBED_DIM * TPAD,), jnp.float32),
        pltpu.VMEM((EMBED_DIM * TPAD,), jnp.float32),
        pltpu.SemaphoreType.DMA,
        pltpu.SemaphoreType.DMA,
        pltpu.SemaphoreType.DMA,
        pltpu.SemaphoreType.DMA,
        pltpu.SemaphoreType.DMA,
    ],
    compiler_params=pltpu.CompilerParams(
        use_tc_tiling_on_sc=False, needs_layout_passes=False),
)
def _gather_kernel(idx_hbm, table_hbm, out_hbm, posa, posb, idxa, idxb,
                   rowsa, rowsb, ta, tb, psema, psemb, gsema, gsemb, osem):
    wid = lax.axis_index("s") * NUM_CORES + lax.axis_index("c")
    iota16 = jax.lax.iota(jnp.int32, 16)
    iota_sc = iota16 * TPAD
    iota_h = iota16 * HIST

    def fetch(k, pos_v, idx_v, rows_v, psem, gsem):
        off = (k * NUM_WORKERS + wid) * CHUNK
        pltpu.sync_copy(idx_hbm.at[pl.ds(off, CHUNK)], idx_v)
        for g in range(CHUNK // 16):
            r = idx_v[pl.ds(g * 16, 16)]
            rl = jax.lax.bitwise_and(r, 511)
            perm = ((r - rl)
                    + jax.lax.shift_left(jax.lax.bitwise_and(r, 15), 5)
                    + jax.lax.shift_right_logical(rl, 4))
            pos_v[pl.ds(g * 16, 16)] = jnp.where(r >= NBLK * TBLK, r, perm)
        return pltpu.async_copy(table_hbm.at[pos_v], rows_v, gsem)

    def process(k, rows_v, tbuf, gsem):
        pltpu.make_async_copy(table_hbm.at[pl.ds(0, CHUNK)], rows_v,
                              gsem).wait()
        @plsc.parallel_loop(0, CHUNK, 1, unroll=8)
        def _transpose(r):
            for q in range(EMBED_DIM // 16):
                vals = rows_v[r, pl.ds(q * 16, 16)]
                plsc.store_scatter(
                    tbuf, [iota_sc + (q * 16 * TPAD) + r], vals)
        u = k * NUM_WORKERS + wid
        h = u // CHUNKS_PER_H
        b0 = (u % CHUNKS_PER_H) * CHUNK
        copies = [
            pltpu.async_copy(
                tbuf.at[pl.ds(c * TPAD, CHUNK)],
                out_hbm.at[h, c, pl.ds(b0, CHUNK)],
                osem,
            )
            for c in range(EMBED_DIM)
        ]
        for cp in copies:
            cp.wait()

    fetch(0, posa, idxa, rowsa, psema, gsema)

    def body(k2, carry):
        ka = 2 * k2
        fetch(ka + 1, posb, idxb, rowsb, psemb, gsemb)
        process(ka, rowsa, ta, gsema)

        @pl.when(k2 + 1 < UNITS_PER_WORKER // 2)
        def _():
            fetch(ka + 2, posa, idxa, rowsa, psema, gsema)

        process(ka + 1, rowsb, tb, gsemb)
        return carry

    lax.fori_loop(0, UNITS_PER_WORKER // 2, body, 0)


def kernel(inputs, table):
    idx_h_major = _idx_kernel(inputs.T)
    table_rm = _table_kernel(table.T).reshape(VOCAB, EMBED_DIM)
    out = _gather_kernel(idx_h_major, table_rm)
    return out.transpose(2, 0, 1)


# speedup vs baseline: 2.7786x; 1.1145x over previous
import functools

import jax
import jax.numpy as jnp
from jax import lax
from jax.experimental import pallas as pl
from jax.experimental.pallas import tpu as pltpu
from jax.experimental.pallas import tpu_sc as plsc

BATCH = 16384
HIST = 50
EMBED_DIM = 32
TOTAL = BATCH * HIST
VOCAB = 1000000

NUM_CORES = 2
NUM_WORKERS = 32

CHUNK = 512
CHUNKS_PER_H = BATCH // CHUNK
NUM_UNITS = TOTAL // CHUNK
UNITS_PER_WORKER = NUM_UNITS // NUM_WORKERS
TPAD = CHUNK + 8

_mesh = plsc.VectorSubcoreMesh(core_axis_name="c", subcore_axis_name="s")


TBLK = 512
NBLK = VOCAB // TBLK
GRP = 1024 + 8


@functools.partial(
    pl.kernel,
    mesh=_mesh,
    out_type=jax.ShapeDtypeStruct((VOCAB * EMBED_DIM,), jnp.float32),
    scratch_types=[
        pltpu.VMEM((EMBED_DIM, TBLK), jnp.float32),
        pltpu.VMEM((16 * GRP,), jnp.float32),
        pltpu.SemaphoreType.DMA,
    ],
    compiler_params=pltpu.CompilerParams(needs_layout_passes=False),
)
def _table_kernel(tt_hbm, flat_hbm, cbuf2d, rbuf, dsem):
    wid = lax.axis_index("s") * NUM_CORES + lax.axis_index("c")
    iota_pb = jax.lax.iota(jnp.int32, 16) * GRP

    def transpose_block(ngrp):
        @plsc.parallel_loop(0, ngrp, 1, unroll=2)
        def _t(g):
            for c in range(EMBED_DIM):
                vals = cbuf2d[c, pl.ds(g * 16, 16)]
                plsc.store_scatter(rbuf, [iota_pb + (g * EMBED_DIM + c)],
                                   vals)

    def writeback(r0, nwords):
        for k in range(16):
            pltpu.async_copy(
                rbuf.at[pl.ds(k * GRP, nwords)],
                flat_hbm.at[pl.ds(r0 * EMBED_DIM + k * 1024, nwords)],
                dsem)
        pltpu.make_async_copy(
            flat_hbm.at[pl.ds(0, 16 * nwords)],
            rbuf.at[pl.ds(0, 16 * nwords)], dsem).wait()

    def body(k, carry):
        j = k * NUM_WORKERS + wid

        @pl.when(j < NBLK)
        def _():
            r0 = j * TBLK
            for kb in range(EMBED_DIM // 8):
                pltpu.async_copy(tt_hbm.at[pl.ds(8 * kb, 8), pl.ds(r0, TBLK)],
                                 cbuf2d.at[pl.ds(8 * kb, 8), :], dsem)
            pltpu.make_async_copy(
                flat_hbm.at[pl.ds(0, EMBED_DIM * TBLK)],
                rbuf.at[pl.ds(0, EMBED_DIM * TBLK)], dsem).wait()
            transpose_block(TBLK // 16)
            writeback(r0, 1024)

        return carry

    lax.fori_loop(0, (NBLK + NUM_WORKERS - 1) // NUM_WORKERS, body, 0)

    @pl.when(wid == NBLK % NUM_WORKERS)
    def _tail():
        r0 = NBLK * TBLK
        tail = VOCAB - r0
        for c in range(EMBED_DIM):
            pltpu.async_copy(tt_hbm.at[c, pl.ds(r0, tail)],
                             cbuf2d.at[c, pl.ds(0, tail)], dsem)
        pltpu.make_async_copy(
            flat_hbm.at[pl.ds(0, EMBED_DIM * tail)],
            rbuf.at[pl.ds(0, EMBED_DIM * tail)], dsem).wait()

        iota32 = jax.lax.iota(jnp.int32, 16) * EMBED_DIM

        @plsc.parallel_loop(0, tail // 16, 1)
        def _tt(g):
            for c in range(EMBED_DIM):
                vals = cbuf2d[c, pl.ds(g * 16, 16)]
                plsc.store_scatter(
                    rbuf, [iota32 + (g * 16 * EMBED_DIM + c)], vals)

        pltpu.sync_copy(rbuf.at[pl.ds(0, tail * EMBED_DIM)],
                        flat_hbm.at[pl.ds(r0 * EMBED_DIM,
                                          tail * EMBED_DIM)])


@functools.partial(
    pl.kernel,
    mesh=_mesh,
    out_type=jax.ShapeDtypeStruct((TOTAL,), jnp.int32),
    scratch_types=[
        pltpu.VMEM((BATCH,), jnp.int32),
    ],
)
def _idx_kernel(idxt_hbm, flat_hbm, vbuf):
    wid = lax.axis_index("s") * NUM_CORES + lax.axis_index("c")

    def move(h):
        pltpu.sync_copy(idxt_hbm.at[h, :], vbuf)
        pltpu.sync_copy(vbuf, flat_hbm.at[pl.ds(h * BATCH, BATCH)])

    move(wid)

    @pl.when(wid < HIST - NUM_WORKERS)
    def _():
        move(wid + NUM_WORKERS)


@functools.partial(
    pl.kernel,
    mesh=_mesh,
    out_type=jax.ShapeDtypeStruct((HIST, EMBED_DIM, BATCH), jnp.float32),
    scratch_types=[
        pltpu.VMEM((CHUNK,), jnp.int32),
        pltpu.VMEM((CHUNK,), jnp.int32),
        pltpu.VMEM((CHUNK,), jnp.int32),
        pltpu.VMEM((CHUNK,), jnp.int32),
        pltpu.VMEM((CHUNK, EMBED_DIM), jnp.float32),
        pltpu.VMEM((CHUNK, EMBED_DIM), jnp.float32),
        pltpu.VMEM((EMBED_DIM * TPAD,), jnp.float32),
        pltpu.VMEM((EMBED_DIM * TPAD,), jnp.float32),
        pltpu.SemaphoreType.DMA,
        pltpu.SemaphoreType.DMA,
        pltpu.SemaphoreType.DMA,
        pltpu.SemaphoreType.DMA,
        pltpu.SemaphoreType.DMA,
    ],
    compiler_params=pltpu.CompilerParams(
        use_tc_tiling_on_sc=False, needs_layout_passes=False),
)
def _gather_kernel(idx_hbm, table_hbm, out_hbm, posa, posb, idxa, idxb,
                   rowsa, rowsb, ta, tb, psema, psemb, gsema, gsemb, osem):
    wid = lax.axis_index("s") * NUM_CORES + lax.axis_index("c")
    iota16 = jax.lax.iota(jnp.int32, 16)
    iota_sc = iota16 * TPAD
    iota_h = iota16 * HIST

    def fetch(k, pos_v, idx_v, rows_v, psem, gsem):
        off = (k * NUM_WORKERS + wid) * CHUNK
        pltpu.sync_copy(idx_hbm.at[pl.ds(off, CHUNK)], idx_v)
        for g in range(CHUNK // 16):
            r = idx_v[pl.ds(g * 16, 16)]
            rl = jax.lax.bitwise_and(r, 511)
            perm = ((r - rl)
                    + jax.lax.shift_left(jax.lax.bitwise_and(r, 15), 5)
                    + jax.lax.shift_right_logical(rl, 4))
            pos_v[pl.ds(g * 16, 16)] = jnp.where(r >= NBLK * TBLK, r, perm)
        return pltpu.async_copy(table_hbm.at[pos_v], rows_v, gsem)

    def process(k, rows_v, tbuf, gsem):
        pltpu.make_async_copy(table_hbm.at[pl.ds(0, CHUNK)], rows_v,
                              gsem).wait()
        @plsc.parallel_loop(0, CHUNK, 1, unroll=8)
        def _transpose(r):
            for q in range(EMBED_DIM // 16):
                vals = rows_v[r, pl.ds(q * 16, 16)]
                plsc.store_scatter(
                    tbuf, [iota_sc + (q * 16 * TPAD) + r], vals)
        u = k * NUM_WORKERS + wid
        h = u // CHUNKS_PER_H
        b0 = (u % CHUNKS_PER_H) * CHUNK
        copies = [
            pltpu.async_copy(
                tbuf.at[pl.ds(c * TPAD, CHUNK)],
                out_hbm.at[h, c, pl.ds(b0, CHUNK)],
                osem,
            )
            for c in range(EMBED_DIM)
        ]
        for cp in copies:
            cp.wait()

    fetch(0, posa, idxa, rowsa, psema, gsema)

    def body(k2, carry):
        ka = 2 * k2
        fetch(ka + 1, posb, idxb, rowsb, psemb, gsemb)
        process(ka, rowsa, ta, gsema)

        @pl.when(k2 + 1 < UNITS_PER_WORKER // 2)
        def _():
            fetch(ka + 2, posa, idxa, rowsa, psema, gsema)

        process(ka + 1, rowsb, tb, gsemb)
        return carry

    lax.fori_loop(0, UNITS_PER_WORKER // 2, body, 0)


def kernel(inputs, table):
    idx_h_major = _idx_kernel(inputs.T)
    table_rm = _table_kernel(table.T).reshape(VOCAB, EMBED_DIM)
    out = _gather_kernel(idx_h_major, table_rm)
    return out.transpose(2, 0, 1)


# speedup vs baseline: 2.8850x; 1.0383x over previous
import functools

import jax
import jax.numpy as jnp
from jax import lax
from jax.experimental import pallas as pl
from jax.experimental.pallas import tpu as pltpu
from jax.experimental.pallas import tpu_sc as plsc

BATCH = 16384
HIST = 50
EMBED_DIM = 32
TOTAL = BATCH * HIST
VOCAB = 1000000

NUM_CORES = 2
NUM_WORKERS = 32

CHUNK = 512
CHUNKS_PER_H = BATCH // CHUNK
NUM_UNITS = TOTAL // CHUNK
UNITS_PER_WORKER = NUM_UNITS // NUM_WORKERS
TPAD = CHUNK + 8

_mesh = plsc.VectorSubcoreMesh(core_axis_name="c", subcore_axis_name="s")


TBLK = 512
NBLK = VOCAB // TBLK
GRP = 1024 + 8


@functools.partial(
    pl.kernel,
    mesh=_mesh,
    out_type=jax.ShapeDtypeStruct((VOCAB * EMBED_DIM,), jnp.float32),
    scratch_types=[
        pltpu.VMEM((EMBED_DIM, TBLK), jnp.float32),
        pltpu.VMEM((16 * GRP,), jnp.float32),
        pltpu.SemaphoreType.DMA,
        pltpu.SemaphoreType.DMA,
    ],
    compiler_params=pltpu.CompilerParams(needs_layout_passes=False),
)
def _table_kernel(tt_hbm, flat_hbm, cbuf2d, rbuf, dsem, wsem):
    wid = lax.axis_index("s") * NUM_CORES + lax.axis_index("c")
    iota_pb = jax.lax.iota(jnp.int32, 16) * GRP

    def transpose_block(ngrp):
        @plsc.parallel_loop(0, ngrp, 1, unroll=2)
        def _t(g):
            for c in range(EMBED_DIM):
                vals = cbuf2d[c, pl.ds(g * 16, 16)]
                plsc.store_scatter(rbuf, [iota_pb + (g * EMBED_DIM + c)],
                                   vals)

    def writeback(r0, nwords):
        for k in range(16):
            pltpu.async_copy(
                rbuf.at[pl.ds(k * GRP, nwords)],
                flat_hbm.at[pl.ds(r0 * EMBED_DIM + k * 1024, nwords)],
                wsem)

    def drain_writeback():
        pltpu.make_async_copy(
            flat_hbm.at[pl.ds(0, 16 * 1024)],
            rbuf.at[pl.ds(0, 16 * 1024)], wsem).wait()

    def body(k, carry):
        j = k * NUM_WORKERS + wid

        @pl.when(j < NBLK)
        def _():
            r0 = j * TBLK
            for kb in range(EMBED_DIM // 8):
                pltpu.async_copy(tt_hbm.at[pl.ds(8 * kb, 8), pl.ds(r0, TBLK)],
                                 cbuf2d.at[pl.ds(8 * kb, 8), :], dsem)
            pltpu.make_async_copy(
                flat_hbm.at[pl.ds(0, EMBED_DIM * TBLK)],
                rbuf.at[pl.ds(0, EMBED_DIM * TBLK)], dsem).wait()

            @pl.when(k > 0)
            def _():
                drain_writeback()

            transpose_block(TBLK // 16)
            writeback(r0, 1024)

        return carry

    lax.fori_loop(0, (NBLK + NUM_WORKERS - 1) // NUM_WORKERS, body, 0)
    drain_writeback()

    @pl.when(wid == NBLK % NUM_WORKERS)
    def _tail():
        r0 = NBLK * TBLK
        tail = VOCAB - r0
        for c in range(EMBED_DIM):
            pltpu.async_copy(tt_hbm.at[c, pl.ds(r0, tail)],
                             cbuf2d.at[c, pl.ds(0, tail)], dsem)
        pltpu.make_async_copy(
            flat_hbm.at[pl.ds(0, EMBED_DIM * tail)],
            rbuf.at[pl.ds(0, EMBED_DIM * tail)], dsem).wait()

        iota32 = jax.lax.iota(jnp.int32, 16) * EMBED_DIM

        @plsc.parallel_loop(0, tail // 16, 1)
        def _tt(g):
            for c in range(EMBED_DIM):
                vals = cbuf2d[c, pl.ds(g * 16, 16)]
                plsc.store_scatter(
                    rbuf, [iota32 + (g * 16 * EMBED_DIM + c)], vals)

        pltpu.sync_copy(rbuf.at[pl.ds(0, tail * EMBED_DIM)],
                        flat_hbm.at[pl.ds(r0 * EMBED_DIM,
                                          tail * EMBED_DIM)])


@functools.partial(
    pl.kernel,
    mesh=_mesh,
    out_type=jax.ShapeDtypeStruct((TOTAL,), jnp.int32),
    scratch_types=[
        pltpu.VMEM((BATCH,), jnp.int32),
    ],
)
def _idx_kernel(idxt_hbm, flat_hbm, vbuf):
    wid = lax.axis_index("s") * NUM_CORES + lax.axis_index("c")

    def move(h):
        pltpu.sync_copy(idxt_hbm.at[h, :], vbuf)
        pltpu.sync_copy(vbuf, flat_hbm.at[pl.ds(h * BATCH, BATCH)])

    move(wid)

    @pl.when(wid < HIST - NUM_WORKERS)
    def _():
        move(wid + NUM_WORKERS)


@functools.partial(
    pl.kernel,
    mesh=_mesh,
    out_type=jax.ShapeDtypeStruct((HIST, EMBED_DIM, BATCH), jnp.float32),
    scratch_types=[
        pltpu.VMEM((CHUNK,), jnp.int32),
        pltpu.VMEM((CHUNK,), jnp.int32),
        pltpu.VMEM((CHUNK,), jnp.int32),
        pltpu.VMEM((CHUNK,), jnp.int32),
        pltpu.VMEM((CHUNK, EMBED_DIM), jnp.float32),
        pltpu.VMEM((CHUNK, EMBED_DIM), jnp.float32),
        pltpu.VMEM((EMBED_DIM * TPAD,), jnp.float32),
        pltpu.VMEM((EMBED_DIM * TPAD,), jnp.float32),
        pltpu.SemaphoreType.DMA,
        pltpu.SemaphoreType.DMA,
        pltpu.SemaphoreType.DMA,
        pltpu.SemaphoreType.DMA,
        pltpu.SemaphoreType.DMA,
    ],
    compiler_params=pltpu.CompilerParams(
        use_tc_tiling_on_sc=False, needs_layout_passes=False),
)
def _gather_kernel(idx_hbm, table_hbm, out_hbm, posa, posb, idxa, idxb,
                   rowsa, rowsb, ta, tb, psema, psemb, gsema, gsemb, osem):
    wid = lax.axis_index("s") * NUM_CORES + lax.axis_index("c")
    iota16 = jax.lax.iota(jnp.int32, 16)
    iota_sc = iota16 * TPAD
    iota_h = iota16 * HIST

    def fetch(k, pos_v, idx_v, rows_v, psem, gsem):
        off = (k * NUM_WORKERS + wid) * CHUNK
        pltpu.sync_copy(idx_hbm.at[pl.ds(off, CHUNK)], idx_v)
        for g in range(CHUNK // 16):
            r = idx_v[pl.ds(g * 16, 16)]
            rl = jax.lax.bitwise_and(r, 511)
            perm = ((r - rl)
                    + jax.lax.shift_left(jax.lax.bitwise_and(r, 15), 5)
                    + jax.lax.shift_right_logical(rl, 4))
            pos_v[pl.ds(g * 16, 16)] = jnp.where(r >= NBLK * TBLK, r, perm)
        return pltpu.async_copy(table_hbm.at[pos_v], rows_v, gsem)

    def process(k, rows_v, tbuf, gsem):
        pltpu.make_async_copy(table_hbm.at[pl.ds(0, CHUNK)], rows_v,
                              gsem).wait()
        @plsc.parallel_loop(0, CHUNK, 1, unroll=8)
        def _transpose(r):
            for q in range(EMBED_DIM // 16):
                vals = rows_v[r, pl.ds(q * 16, 16)]
                plsc.store_scatter(
                    tbuf, [iota_sc + (q * 16 * TPAD) + r], vals)
        u = k * NUM_WORKERS + wid
        h = u // CHUNKS_PER_H
        b0 = (u % CHUNKS_PER_H) * CHUNK
        copies = [
            pltpu.async_copy(
                tbuf.at[pl.ds(c * TPAD, CHUNK)],
                out_hbm.at[h, c, pl.ds(b0, CHUNK)],
                osem,
            )
            for c in range(EMBED_DIM)
        ]
        for cp in copies:
            cp.wait()

    fetch(0, posa, idxa, rowsa, psema, gsema)

    def body(k2, carry):
        ka = 2 * k2
        fetch(ka + 1, posb, idxb, rowsb, psemb, gsemb)
        process(ka, rowsa, ta, gsema)

        @pl.when(k2 + 1 < UNITS_PER_WORKER // 2)
        def _():
            fetch(ka + 2, posa, idxa, rowsa, psema, gsema)

        process(ka + 1, rowsb, tb, gsemb)
        return carry

    lax.fori_loop(0, UNITS_PER_WORKER // 2, body, 0)


def kernel(inputs, table):
    idx_h_major = _idx_kernel(inputs.T)
    table_rm = _table_kernel(table.T).reshape(VOCAB, EMBED_DIM)
    out = _gather_kernel(idx_h_major, table_rm)
    return out.transpose(2, 0, 1)
